# Initial kernel scaffold; baseline (speedup 1.0000x reference)
import jax
import jax.numpy as jnp
from jax.experimental import pallas as pl


def kernel(rays_o, rays_d, vox_idx, t_near, t_far):
    raise NotImplementedError("write your pallas kernel here")


# speedup vs baseline: 2361.6701x; 2361.6701x over previous
import jax
import jax.numpy as jnp
from jax.experimental import pallas as pl

_N_RAYS = 65536
_MAX_HITS = 32
_FIXED = 128
_MAX_STEPS = 160
_MAX_DEPTH = 10000.0
_BLOCK_R = 256


def _cumsum_lanes(x, n):
    sh = 1
    while sh < n:
        x = x + jnp.concatenate([jnp.zeros_like(x[:, :sh]), x[:, :-sh]], axis=1)
        sh *= 2
    return x


def _body(ro_ref, rdir_ref, vi_ref, tn_ref, tf_ref,
          pts_ref, vout_ref, dout_ref, sout_ref):
    tn = tn_ref[...]
    tf = tf_ref[...]
    vi = vi_ref[...]
    R = tn.shape[0]

    rng = jnp.where(vi == -1, 0.0, tf - tn)
    total = jnp.sum(rng, axis=1, keepdims=True)
    prob = rng / total
    cdf = _cumsum_lanes(prob, _MAX_HITS)
    pclip = jnp.maximum(prob, 1e-12)
    s = (tf - tn) / pclip
    cdf_prev = jnp.concatenate([jnp.zeros_like(cdf[:, :1]), cdf[:, :-1]], axis=1)
    c = tn - cdf_prev * s

    u = (jax.lax.broadcasted_iota(jnp.int32, (1, _FIXED), 1).astype(jnp.float32)
         + 0.5) * (1.0 / _FIXED)
    c_g = jnp.broadcast_to(c[:, 0:1], (R, _FIXED))
    s_g = jnp.broadcast_to(s[:, 0:1], (R, _FIXED))
    v_g = jnp.broadcast_to(vi[:, 0:1], (R, _FIXED))
    for k in range(_MAX_HITS - 1):
        ind = u >= cdf[:, k:k + 1]
        c_g = jnp.where(ind, c[:, k + 1:k + 2], c_g)
        s_g = jnp.where(ind, s[:, k + 1:k + 2], s_g)
        v_g = jnp.where(ind, vi[:, k + 1:k + 2], v_g)
    t_raw = c_g + s_g * u

    u128 = (_FIXED + 0.5) / _FIXED
    t128 = c[:, _MAX_HITS - 1:] + s[:, _MAX_HITS - 1:] * u128
    nxt = jnp.concatenate([t_raw[:, 1:], t128], axis=1)
    prv = jnp.concatenate([t_raw[:, :1], t_raw[:, :-1]], axis=1)
    dist = jnp.maximum((nxt - prv) * 0.5, 0.0)

    tail = _MAX_STEPS - _FIXED
    dout_ref[:, :_FIXED] = t_raw
    dout_ref[:, _FIXED:] = jnp.full((R, tail), _MAX_DEPTH, jnp.float32)
    vout_ref[:, :_FIXED] = v_g
    vout_ref[:, _FIXED:] = jnp.full((R, tail), -1, jnp.int32)
    sout_ref[:, :_FIXED] = dist
    sout_ref[:, _FIXED:] = jnp.zeros((R, tail), jnp.float32)

    W = 3 * _MAX_STEPS
    Wh = 3 * _FIXED
    mod3 = jax.lax.broadcasted_iota(jnp.int32, (1, W), 1) % 3
    ro = ro_ref[...]
    rdir = rdir_ref[...]
    o_il = jnp.where(mod3 == 0, ro[:, 0:1],
                     jnp.where(mod3 == 1, ro[:, 1:2], ro[:, 2:3]))
    d_il = jnp.where(mod3 == 0, rdir[:, 0:1],
                     jnp.where(mod3 == 1, rdir[:, 1:2], rdir[:, 2:3]))
    jj = jax.lax.broadcasted_iota(jnp.int32, (_FIXED, Wh), 0)
    ii = jax.lax.broadcasted_iota(jnp.int32, (_FIXED, Wh), 1)
    expand = (ii // 3 == jj).astype(jnp.float32)
    t_il = jnp.dot(t_raw, expand, preferred_element_type=jnp.float32)
    pts_ref[:, :Wh] = o_il[:, :Wh] + t_il * d_il[:, :Wh]
    pts_ref[:, Wh:] = o_il[:, Wh:] + _MAX_DEPTH * d_il[:, Wh:]


def kernel(rays_o, rays_d, vox_idx, t_near, t_far):
    n = rays_o.shape[0]
    grid = (n // _BLOCK_R,)
    row = lambda i: (i, 0)
    pts_il, vidx, depth, dists = pl.pallas_call(
        _body,
        grid=grid,
        in_specs=[
            pl.BlockSpec((_BLOCK_R, 3), row),
            pl.BlockSpec((_BLOCK_R, 3), row),
            pl.BlockSpec((_BLOCK_R, _MAX_HITS), row),
            pl.BlockSpec((_BLOCK_R, _MAX_HITS), row),
            pl.BlockSpec((_BLOCK_R, _MAX_HITS), row),
        ],
        out_specs=[
            pl.BlockSpec((_BLOCK_R, 3 * _MAX_STEPS), row),
            pl.BlockSpec((_BLOCK_R, _MAX_STEPS), row),
            pl.BlockSpec((_BLOCK_R, _MAX_STEPS), row),
            pl.BlockSpec((_BLOCK_R, _MAX_STEPS), row),
        ],
        out_shape=[
            jax.ShapeDtypeStruct((n, 3 * _MAX_STEPS), jnp.float32),
            jax.ShapeDtypeStruct((n, _MAX_STEPS), jnp.int32),
            jax.ShapeDtypeStruct((n, _MAX_STEPS), jnp.float32),
            jax.ShapeDtypeStruct((n, _MAX_STEPS), jnp.float32),
        ],
    )(rays_o, rays_d, vox_idx, t_near, t_far)
    pts = pts_il.reshape(n, _MAX_STEPS, 3)
    return (pts, vidx, depth, dists)


# speedup vs baseline: 4851.9001x; 2.0544x over previous
import jax
import jax.numpy as jnp
from jax.experimental import pallas as pl

_MAX_HITS = 32
_FIXED = 128
_MAX_STEPS = 160
_MAX_DEPTH = 10000.0
_BLOCK_R = 128

_DN0 = (((0,), (0,)), ((), ()))


def _cumsum_sub(x, n):
    sh = 1
    while sh < n:
        x = x + jnp.concatenate([jnp.zeros_like(x[:sh]), x[:-sh]], axis=0)
        sh *= 2
    return x


def _body(ro_ref, rdir_ref, vi_ref, tn_ref, tf_ref,
          pts_ref, vout_ref, dout_ref, sout_ref):
    tn = tn_ref[...]
    tf = tf_ref[...]
    vi = vi_ref[...]
    vif = vi.astype(jnp.float32)
    R = tn.shape[1]

    rng = jnp.where(vi == -1, 0.0, tf - tn)
    total = jnp.sum(rng, axis=0, keepdims=True)
    prob = rng / total
    cdf = _cumsum_sub(prob, _MAX_HITS)
    pclip = jnp.maximum(prob, 1e-12)
    s = (tf - tn) / pclip
    cdf_prev = jnp.concatenate([jnp.zeros_like(cdf[:1]), cdf[:-1]], axis=0)
    c = tn - cdf_prev * s

    u = (jax.lax.broadcasted_iota(jnp.int32, (_FIXED, R), 0).astype(jnp.float32)
         + 0.5) * (1.0 / _FIXED)
    c_g = jnp.broadcast_to(c[0:1], (_FIXED, R))
    s_g = jnp.broadcast_to(s[0:1], (_FIXED, R))
    v_g = jnp.broadcast_to(vif[0:1], (_FIXED, R))
    for k in range(_MAX_HITS - 1):
        ind = u >= cdf[k:k + 1]
        c_g = jnp.where(ind, c[k + 1:k + 2], c_g)
        s_g = jnp.where(ind, s[k + 1:k + 2], s_g)
        v_g = jnp.where(ind, vif[k + 1:k + 2], v_g)
    t_raw = c_g + s_g * u

    u128 = (_FIXED + 0.5) / _FIXED
    t128 = c[_MAX_HITS - 1:] + s[_MAX_HITS - 1:] * u128
    nxt = jnp.concatenate([t_raw[1:], t128], axis=0)
    prv = jnp.concatenate([t_raw[:1], t_raw[:-1]], axis=0)
    dist = jnp.maximum((nxt - prv) * 0.5, 0.0)

    jj = jax.lax.broadcasted_iota(jnp.int32, (_FIXED, _FIXED), 0)
    cc = jax.lax.broadcasted_iota(jnp.int32, (_FIXED, _FIXED), 1)
    eye = (jj == cc).astype(jnp.float32)
    depth = jax.lax.dot_general(t_raw, eye, _DN0,
                                preferred_element_type=jnp.float32)
    v_out = jax.lax.dot_general(v_g, eye, _DN0,
                                preferred_element_type=jnp.float32)
    dist_out = jax.lax.dot_general(dist, eye, _DN0,
                                   preferred_element_type=jnp.float32)

    tail = _MAX_STEPS - _FIXED
    dout_ref[:, :_FIXED] = depth
    dout_ref[:, _FIXED:] = jnp.full((R, tail), _MAX_DEPTH, jnp.float32)
    vout_ref[:, :_FIXED] = v_out.astype(jnp.int32)
    vout_ref[:, _FIXED:] = jnp.full((R, tail), -1, jnp.int32)
    sout_ref[:, :_FIXED] = dist_out
    sout_ref[:, _FIXED:] = jnp.zeros((R, tail), jnp.float32)

    W = 3 * _MAX_STEPS
    Wh = 3 * _FIXED
    je = jax.lax.broadcasted_iota(jnp.int32, (_FIXED, Wh), 0)
    ie = jax.lax.broadcasted_iota(jnp.int32, (_FIXED, Wh), 1)
    expand = (ie // 3 == je).astype(jnp.float32)
    t_il = jax.lax.dot_general(t_raw, expand, _DN0,
                               preferred_element_type=jnp.float32)
    mod3 = jax.lax.broadcasted_iota(jnp.int32, (3, W), 1) % 3
    ax3 = jax.lax.broadcasted_iota(jnp.int32, (3, W), 0)
    sel3 = (mod3 == ax3).astype(jnp.float32)
    o_il = jnp.dot(ro_ref[...], sel3, preferred_element_type=jnp.float32)
    d_il = jnp.dot(rdir_ref[...], sel3, preferred_element_type=jnp.float32)
    pts_ref[:, :Wh] = o_il[:, :Wh] + t_il * d_il[:, :Wh]
    pts_ref[:, Wh:] = o_il[:, Wh:] + _MAX_DEPTH * d_il[:, Wh:]


def kernel(rays_o, rays_d, vox_idx, t_near, t_far):
    n = rays_o.shape[0]
    grid = (n // _BLOCK_R,)
    row = lambda i: (i, 0)
    col = lambda i: (0, i)
    pts_il, vidx, depth, dists = pl.pallas_call(
        _body,
        grid=grid,
        in_specs=[
            pl.BlockSpec((_BLOCK_R, 3), row),
            pl.BlockSpec((_BLOCK_R, 3), row),
            pl.BlockSpec((_MAX_HITS, _BLOCK_R), col),
            pl.BlockSpec((_MAX_HITS, _BLOCK_R), col),
            pl.BlockSpec((_MAX_HITS, _BLOCK_R), col),
        ],
        out_specs=[
            pl.BlockSpec((_BLOCK_R, 3 * _MAX_STEPS), row),
            pl.BlockSpec((_BLOCK_R, _MAX_STEPS), row),
            pl.BlockSpec((_BLOCK_R, _MAX_STEPS), row),
            pl.BlockSpec((_BLOCK_R, _MAX_STEPS), row),
        ],
        out_shape=[
            jax.ShapeDtypeStruct((n, 3 * _MAX_STEPS), jnp.float32),
            jax.ShapeDtypeStruct((n, _MAX_STEPS), jnp.int32),
            jax.ShapeDtypeStruct((n, _MAX_STEPS), jnp.float32),
            jax.ShapeDtypeStruct((n, _MAX_STEPS), jnp.float32),
        ],
    )(rays_o, rays_d, vox_idx.T, t_near.T, t_far.T)
    pts = pts_il.reshape(n, _MAX_STEPS, 3)
    return (pts, vidx, depth, dists)
